# Initial kernel scaffold; baseline (speedup 1.0000x reference)
import functools

import jax
import jax.numpy as jnp
from jax import lax
from jax.experimental import pallas as pl
from jax.experimental.pallas import tpu as pltpu
from jax.experimental.pallas import tpu_sc as plsc

N = 10000
SEQ = 6
DIM = 3
LATENT = 64
NTYPES = 9
RADIUS = 0.5
E = 320000

NP_ = 10240
NW = 32
CHUNK = 128
KCH = 79
EP = NW * KCH * CHUNK
BE = 1024
BN = 256
ROWS_PER_TILE = NP_ // 16


def _ln(y):
    m = jnp.mean(y, axis=-1, keepdims=True)
    d = y - m
    v = jnp.mean(d * d, axis=-1, keepdims=True)
    return d / jnp.sqrt(v + 1e-5)


def _dot(a, b):
    return jnp.dot(a, b, preferred_element_type=jnp.float32)


def _relu(x):
    return jnp.maximum(x, 0.0)


def _rows(bs, ncols):
    return pl.BlockSpec((bs, ncols), lambda i: (i, 0))


def _const(shape):
    return pl.BlockSpec(shape, lambda i: tuple(0 for _ in shape))


def _enc_node_body(pf_ref, tt_ref, emb_ref, w1, b1, w2, b2, w3, b3, out_ref):
    pf = pf_ref[...]
    vel = pf[:, 3:18] - pf[:, 0:15]
    mr = pf[:, 15:18]
    ncb = jnp.clip(
        jnp.concatenate([mr, 1.0 - mr], axis=1) * (1.0 / RADIUS), -1.0, 1.0)
    oh = (lax.broadcasted_iota(jnp.int32, (BN, 16), 1) == tt_ref[...]).astype(
        jnp.float32)
    embv = _dot(oh, emb_ref[...])
    x = jnp.concatenate([vel, ncb, embv], axis=1)
    h = _relu(_dot(x, w1[...]) + b1[...])
    h = _relu(_dot(h, w2[...]) + b2[...])
    out_ref[...] = _ln(_dot(h, w3[...]) + b3[...])


def _enc_edge_body(ps_ref, pr_ref, w1, b1, w2, b2, w3, b3, out_ref):
    d = (ps_ref[...] - pr_ref[...]) * (1.0 / RADIUS)
    dist = jnp.sqrt(jnp.sum(d * d, axis=-1, keepdims=True))
    x = jnp.concatenate([d[:, 0:3], dist], axis=1)
    h = _relu(_dot(x, w1[...]) + b1[...])
    h = _relu(_dot(h, w2[...]) + b2[...])
    out_ref[...] = _ln(_dot(h, w3[...]) + b3[...])


def _edge_step_body(el_ref, gs_ref, gr_ref, w1e, w1s, w1r, b1, w2, b2, w3, b3,
                    eupd_ref, elnew_ref):
    el = el_ref[...]
    z = (_dot(el, w1e[...]) + _dot(gs_ref[...], w1s[...])
         + _dot(gr_ref[...], w1r[...]) + b1[...])
    h = _relu(z)
    h = _relu(_dot(h, w2[...]) + b2[...])
    e = _ln(_dot(h, w3[...]) + b3[...])
    eupd_ref[...] = e
    elnew_ref[...] = el + e


def _node_step_body(nl_ref, a0_ref, a1_ref, w1n, w1a, b1, w2, b2, w3, b3,
                    out_ref):
    nl = nl_ref[...]
    agg = a0_ref[0] + a1_ref[0]
    z = _dot(nl, w1n[...]) + _dot(agg, w1a[...]) + b1[...]
    h = _relu(z)
    h = _relu(_dot(h, w2[...]) + b2[...])
    out_ref[...] = nl + _ln(_dot(h, w3[...]) + b3[...])


def _dec_body(nl_ref, pf_ref, w1, b1, w2, b2, w3, b3, out_ref):
    h = _relu(_dot(nl_ref[...], w1[...]) + b1[...])
    h = _relu(_dot(h, w2[...]) + b2[...])
    acc = _dot(h, w3[...]) + b3[...]
    pf = pf_ref[...]
    mr = pf[:, 15:18]
    prev = pf[:, 12:15]
    out_ref[...] = 2.0 * mr - prev + acc


def _mlp_specs(sizes):
    specs = []
    for i in range(len(sizes) - 1):
        specs.append(_const((sizes[i], sizes[i + 1])))
        specs.append(_const((1, sizes[i + 1])))
    return specs


@functools.lru_cache(maxsize=None)
def _enc_node_call():
    return pl.pallas_call(
        _enc_node_body,
        grid=(NP_ // BN,),
        in_specs=[_rows(BN, 18), _rows(BN, 1), _const((16, 16))]
        + _mlp_specs([37, 64, 64, 64]),
        out_specs=_rows(BN, LATENT),
        out_shape=jax.ShapeDtypeStruct((NP_, LATENT), jnp.float32),
    )


@functools.lru_cache(maxsize=None)
def _enc_edge_call():
    return pl.pallas_call(
        _enc_edge_body,
        grid=(EP // BE,),
        in_specs=[_rows(BE, 8), _rows(BE, 8)] + _mlp_specs([4, 64, 64, 64]),
        out_specs=_rows(BE, LATENT),
        out_shape=jax.ShapeDtypeStruct((EP, LATENT), jnp.float32),
    )


@functools.lru_cache(maxsize=None)
def _edge_step_call():
    return pl.pallas_call(
        _edge_step_body,
        grid=(EP // BE,),
        in_specs=[_rows(BE, LATENT)] * 3
        + [_const((64, 64)), _const((64, 64)), _const((64, 64)),
           _const((1, 64)),
           _const((64, 64)), _const((1, 64)), _const((64, 64)),
           _const((1, 64))],
        out_specs=(_rows(BE, LATENT), _rows(BE, LATENT)),
        out_shape=(jax.ShapeDtypeStruct((EP, LATENT), jnp.float32),
                   jax.ShapeDtypeStruct((EP, LATENT), jnp.float32)),
    )


@functools.lru_cache(maxsize=None)
def _node_step_call():
    return pl.pallas_call(
        _node_step_body,
        grid=(NP_ // BN,),
        in_specs=[_rows(BN, LATENT),
                  pl.BlockSpec((1, BN, LATENT), lambda i: (0, i, 0)),
                  pl.BlockSpec((1, BN, LATENT), lambda i: (1, i, 0)),
                  _const((64, 64)), _const((64, 64)), _const((1, 64)),
                  _const((64, 64)), _const((1, 64)), _const((64, 64)),
                  _const((1, 64))],
        out_specs=_rows(BN, LATENT),
        out_shape=jax.ShapeDtypeStruct((NP_, LATENT), jnp.float32),
    )


@functools.lru_cache(maxsize=None)
def _dec_call():
    return pl.pallas_call(
        _dec_body,
        grid=(NP_ // BN,),
        in_specs=[_rows(BN, LATENT), _rows(BN, 18)]
        + _mlp_specs([64, 64, 64, 3]),
        out_specs=_rows(BN, 3),
        out_shape=jax.ShapeDtypeStruct((NP_, 3), jnp.float32),
    )


@functools.lru_cache(maxsize=None)
def _mesh():
    return plsc.VectorSubcoreMesh(
        core_axis_name="c", subcore_axis_name="s", num_cores=2,
        num_subcores=16)


@functools.lru_cache(maxsize=None)
def _gather_call(d):
    @functools.partial(
        pl.kernel,
        out_type=(jax.ShapeDtypeStruct((EP, d), jnp.float32),
                  jax.ShapeDtypeStruct((EP, d), jnp.float32)),
        mesh=_mesh(),
        scratch_types=[
            pltpu.VMEM((KCH, CHUNK), jnp.int32),
            pltpu.VMEM((KCH, CHUNK), jnp.int32),
            pltpu.VMEM((CHUNK, d), jnp.float32),
            pltpu.VMEM((CHUNK, d), jnp.float32),
            pltpu.SemaphoreType.DMA,
            pltpu.SemaphoreType.DMA,
        ],
    )
    def gath(tab_hbm, idxs_hbm, idxr_hbm, outs_hbm, outr_hbm,
             idxs_v, idxr_v, bufs, bufr, sems, semr):
        wid = lax.axis_index("s") * 2 + lax.axis_index("c")
        row0 = wid * KCH
        pltpu.sync_copy(idxs_hbm.at[pl.ds(row0, KCH)], idxs_v)
        pltpu.sync_copy(idxr_hbm.at[pl.ds(row0, KCH)], idxr_v)

        def body(j, carry):
            base = (row0 + j) * CHUNK
            cs = pltpu.async_copy(tab_hbm.at[idxs_v.at[j]], bufs, sems)
            cr = pltpu.async_copy(tab_hbm.at[idxr_v.at[j]], bufr, semr)
            cs.wait()
            pltpu.sync_copy(bufs, outs_hbm.at[pl.ds(base, CHUNK)])
            cr.wait()
            pltpu.sync_copy(bufr, outr_hbm.at[pl.ds(base, CHUNK)])
            return carry

        lax.fori_loop(0, KCH, body, 0)

    return gath


@functools.lru_cache(maxsize=None)
def _scatter_call():
    @functools.partial(
        pl.kernel,
        out_type=jax.ShapeDtypeStruct((2, NP_, LATENT), jnp.float32),
        mesh=_mesh(),
        scratch_types=[
            pltpu.VMEM((KCH, CHUNK), jnp.int32),
            pltpu.VMEM((CHUNK, LATENT), jnp.float32),
            pltpu.VMEM_SHARED((NP_, LATENT), jnp.float32),
            pltpu.SemaphoreType.DMA,
        ],
    )
    def scat(idxr_hbm, val_hbm, zero_hbm, out_hbm, idx_v, buf, shared, sem):
        c = lax.axis_index("c")
        s = lax.axis_index("s")
        wid = s * 2 + c
        z0 = s * ROWS_PER_TILE
        pltpu.sync_copy(zero_hbm.at[pl.ds(z0, ROWS_PER_TILE)],
                        shared.at[pl.ds(z0, ROWS_PER_TILE)])
        plsc.subcore_barrier()
        row0 = wid * KCH
        pltpu.sync_copy(idxr_hbm.at[pl.ds(row0, KCH)], idx_v)

        def body(j, carry):
            base = (row0 + j) * CHUNK
            pltpu.sync_copy(val_hbm.at[pl.ds(base, CHUNK)], buf)
            pltpu.sync_copy(buf, shared.at[idx_v.at[j]], add=True)
            return carry

        lax.fori_loop(0, KCH, body, 0)
        plsc.subcore_barrier()
        pltpu.sync_copy(shared.at[pl.ds(z0, ROWS_PER_TILE)],
                        out_hbm.at[c, pl.ds(z0, ROWS_PER_TILE)])

    return scat


def _mlp_args(ps):
    out = []
    for (w, b) in ps:
        out.append(w)
        out.append(b.reshape(1, -1))
    return out


def kernel(position_sequence, n_particles_per_example, particle_types,
           senders, receivers, particle_type_embedding, params):
    del n_particles_per_example
    pf = position_sequence.reshape(N, SEQ * DIM)
    pf_p = jnp.pad(pf, ((0, NP_ - N), (0, 0)))
    tt_p = jnp.pad(particle_types.astype(jnp.int32),
                   (0, NP_ - N)).reshape(NP_, 1)
    emb_p = jnp.pad(particle_type_embedding, ((0, 16 - NTYPES), (0, 0)))
    snd_p = jnp.pad(senders.astype(jnp.int32),
                    (0, EP - E)).reshape(EP // CHUNK, CHUNK)
    rcv_p = jnp.pad(receivers.astype(jnp.int32), (0, EP - E),
                    constant_values=N).reshape(EP // CHUNK, CHUNK)
    postab = jnp.pad(pf[:, 15:18], ((0, NP_ - N), (0, 5)))
    zeros_n = jnp.zeros((NP_, LATENT), jnp.float32)

    nlat = _enc_node_call()(pf_p, tt_p, emb_p, *_mlp_args(params['enc_node']))
    ps, pr = _gather_call(8)(postab, snd_p, rcv_p)
    elat = _enc_edge_call()(ps, pr, *_mlp_args(params['enc_edge']))

    for s in range(5):
        gs, gr = _gather_call(64)(nlat, snd_p, rcv_p)
        (w1, b1), (w2, b2), (w3, b3) = params['proc_edge'][s]
        eupd, elat = _edge_step_call()(
            elat, gs, gr, w1[0:64], w1[64:128], w1[128:192],
            b1.reshape(1, -1), w2, b2.reshape(1, -1), w3, b3.reshape(1, -1))
        aggp = _scatter_call()(rcv_p, eupd, zeros_n)
        (v1, c1), (v2, c2), (v3, c3) = params['proc_node'][s]
        nlat = _node_step_call()(
            nlat, aggp, aggp, v1[0:64], v1[64:128], c1.reshape(1, -1),
            v2, c2.reshape(1, -1), v3, c3.reshape(1, -1))

    out = _dec_call()(nlat, pf_p, *_mlp_args(params['dec']))
    return out[:N]


# speedup vs baseline: 1.6689x; 1.6689x over previous
import functools

import jax
import jax.numpy as jnp
from jax import lax
from jax.experimental import pallas as pl
from jax.experimental.pallas import tpu as pltpu
from jax.experimental.pallas import tpu_sc as plsc

N = 10000
SEQ = 6
DIM = 3
LATENT = 64
NTYPES = 9
RADIUS = 0.5
E = 320000

NP_ = 10240
NW = 32
CHUNK = 128
KCH = 80
EP = NW * KCH * CHUNK
BE = 1024
BN = 256
ROWS_PER_TILE = NP_ // 16


def _ln(y):
    m = jnp.mean(y, axis=-1, keepdims=True)
    d = y - m
    v = jnp.mean(d * d, axis=-1, keepdims=True)
    return d / jnp.sqrt(v + 1e-5)


def _dot(a, b):
    return jnp.dot(a, b, preferred_element_type=jnp.float32)


def _relu(x):
    return jnp.maximum(x, 0.0)


def _rows(bs, ncols):
    return pl.BlockSpec((bs, ncols), lambda i: (i, 0))


def _const(shape):
    return pl.BlockSpec(shape, lambda i: tuple(0 for _ in shape))


def _enc_node_body(pf_ref, tt_ref, emb_ref, w1, b1, w2, b2, w3, b3, out_ref):
    pf = pf_ref[...]
    vel = pf[:, 3:18] - pf[:, 0:15]
    mr = pf[:, 15:18]
    ncb = jnp.clip(
        jnp.concatenate([mr, 1.0 - mr], axis=1) * (1.0 / RADIUS), -1.0, 1.0)
    oh = (lax.broadcasted_iota(jnp.int32, (BN, 16), 1) == tt_ref[...]).astype(
        jnp.float32)
    embv = _dot(oh, emb_ref[...])
    x = jnp.concatenate([vel, ncb, embv], axis=1)
    h = _relu(_dot(x, w1[...]) + b1[...])
    h = _relu(_dot(h, w2[...]) + b2[...])
    out_ref[...] = _ln(_dot(h, w3[...]) + b3[...])


def _enc_edge_body(ps_ref, pr_ref, w1, b1, w2, b2, w3, b3, out_ref):
    d = (ps_ref[...] - pr_ref[...]) * (1.0 / RADIUS)
    dist = jnp.sqrt(jnp.sum(d * d, axis=-1, keepdims=True))
    x = jnp.concatenate([d[:, 0:3], dist], axis=1)
    h = _relu(_dot(x, w1[...]) + b1[...])
    h = _relu(_dot(h, w2[...]) + b2[...])
    out_ref[...] = _ln(_dot(h, w3[...]) + b3[...])


def _edge_step_body(el_ref, gs_ref, gr_ref, w1e, w1s, w1r, b1, w2, b2, w3, b3,
                    eupd_ref, elnew_ref):
    el = el_ref[...]
    z = (_dot(el, w1e[...]) + _dot(gs_ref[...], w1s[...])
         + _dot(gr_ref[...], w1r[...]) + b1[...])
    h = _relu(z)
    h = _relu(_dot(h, w2[...]) + b2[...])
    e = _ln(_dot(h, w3[...]) + b3[...])
    eupd_ref[...] = e
    elnew_ref[...] = el + e


def _node_step_body(nl_ref, a0_ref, a1_ref, w1n, w1a, b1, w2, b2, w3, b3,
                    out_ref):
    nl = nl_ref[...]
    agg = a0_ref[0] + a1_ref[0]
    z = _dot(nl, w1n[...]) + _dot(agg, w1a[...]) + b1[...]
    h = _relu(z)
    h = _relu(_dot(h, w2[...]) + b2[...])
    out_ref[...] = nl + _ln(_dot(h, w3[...]) + b3[...])


def _dec_body(nl_ref, pf_ref, w1, b1, w2, b2, w3, b3, out_ref):
    h = _relu(_dot(nl_ref[...], w1[...]) + b1[...])
    h = _relu(_dot(h, w2[...]) + b2[...])
    acc = _dot(h, w3[...]) + b3[...]
    pf = pf_ref[...]
    mr = pf[:, 15:18]
    prev = pf[:, 12:15]
    out_ref[...] = 2.0 * mr - prev + acc


def _mlp_specs(sizes):
    specs = []
    for i in range(len(sizes) - 1):
        specs.append(_const((sizes[i], sizes[i + 1])))
        specs.append(_const((1, sizes[i + 1])))
    return specs


@functools.lru_cache(maxsize=None)
def _enc_node_call():
    return pl.pallas_call(
        _enc_node_body,
        grid=(NP_ // BN,),
        in_specs=[_rows(BN, 18), _rows(BN, 1), _const((16, 16))]
        + _mlp_specs([37, 64, 64, 64]),
        out_specs=_rows(BN, LATENT),
        out_shape=jax.ShapeDtypeStruct((NP_, LATENT), jnp.float32),
    )


@functools.lru_cache(maxsize=None)
def _enc_edge_call():
    return pl.pallas_call(
        _enc_edge_body,
        grid=(EP // BE,),
        in_specs=[_rows(BE, 8), _rows(BE, 8)] + _mlp_specs([4, 64, 64, 64]),
        out_specs=_rows(BE, LATENT),
        out_shape=jax.ShapeDtypeStruct((EP, LATENT), jnp.float32),
    )


@functools.lru_cache(maxsize=None)
def _edge_step_call():
    return pl.pallas_call(
        _edge_step_body,
        grid=(EP // BE,),
        in_specs=[_rows(BE, LATENT)] * 3
        + [_const((64, 64)), _const((64, 64)), _const((64, 64)),
           _const((1, 64)),
           _const((64, 64)), _const((1, 64)), _const((64, 64)),
           _const((1, 64))],
        out_specs=(_rows(BE, LATENT), _rows(BE, LATENT)),
        out_shape=(jax.ShapeDtypeStruct((EP, LATENT), jnp.float32),
                   jax.ShapeDtypeStruct((EP, LATENT), jnp.float32)),
    )


@functools.lru_cache(maxsize=None)
def _node_step_call():
    return pl.pallas_call(
        _node_step_body,
        grid=(NP_ // BN,),
        in_specs=[_rows(BN, LATENT),
                  pl.BlockSpec((1, BN, LATENT), lambda i: (0, i, 0)),
                  pl.BlockSpec((1, BN, LATENT), lambda i: (1, i, 0)),
                  _const((64, 64)), _const((64, 64)), _const((1, 64)),
                  _const((64, 64)), _const((1, 64)), _const((64, 64)),
                  _const((1, 64))],
        out_specs=_rows(BN, LATENT),
        out_shape=jax.ShapeDtypeStruct((NP_, LATENT), jnp.float32),
    )


@functools.lru_cache(maxsize=None)
def _dec_call():
    return pl.pallas_call(
        _dec_body,
        grid=(NP_ // BN,),
        in_specs=[_rows(BN, LATENT), _rows(BN, 18)]
        + _mlp_specs([64, 64, 64, 3]),
        out_specs=_rows(BN, 3),
        out_shape=jax.ShapeDtypeStruct((NP_, 3), jnp.float32),
    )


@functools.lru_cache(maxsize=None)
def _mesh():
    return plsc.VectorSubcoreMesh(
        core_axis_name="c", subcore_axis_name="s", num_cores=2,
        num_subcores=16)


_SC_PARAMS = pltpu.CompilerParams(use_tc_tiling_on_sc=False)


@functools.lru_cache(maxsize=None)
def _gather_call(d):
    @functools.partial(
        pl.kernel,
        out_type=(jax.ShapeDtypeStruct((EP, d), jnp.float32),
                  jax.ShapeDtypeStruct((EP, d), jnp.float32)),
        mesh=_mesh(),
        scratch_types=[
            pltpu.VMEM((KCH, CHUNK), jnp.int32),
            pltpu.VMEM((KCH, CHUNK), jnp.int32),
            pltpu.VMEM((CHUNK, d), jnp.float32),
            pltpu.VMEM((CHUNK, d), jnp.float32),
            pltpu.SemaphoreType.DMA,
            pltpu.SemaphoreType.DMA,
        ],
        compiler_params=_SC_PARAMS,
    )
    def gath(tab_hbm, idxs_hbm, idxr_hbm, outs_hbm, outr_hbm,
             idxs_v, idxr_v, bufs, bufr, sems, semr):
        wid = lax.axis_index("s") * 2 + lax.axis_index("c")
        row0 = wid * KCH
        pltpu.sync_copy(idxs_hbm.at[pl.ds(row0, KCH)], idxs_v)
        pltpu.sync_copy(idxr_hbm.at[pl.ds(row0, KCH)], idxr_v)

        def body(j, carry):
            base = (row0 + j) * CHUNK
            cs = pltpu.async_copy(tab_hbm.at[idxs_v.at[j]], bufs, sems)
            cr = pltpu.async_copy(tab_hbm.at[idxr_v.at[j]], bufr, semr)
            cs.wait()
            pltpu.sync_copy(bufs, outs_hbm.at[pl.ds(base, CHUNK)])
            cr.wait()
            pltpu.sync_copy(bufr, outr_hbm.at[pl.ds(base, CHUNK)])
            return carry

        lax.fori_loop(0, KCH, body, 0)

    return gath


@functools.lru_cache(maxsize=None)
def _scatter_call():
    @functools.partial(
        pl.kernel,
        out_type=jax.ShapeDtypeStruct((2, NP_, LATENT), jnp.float32),
        mesh=_mesh(),
        scratch_types=[
            pltpu.VMEM((KCH, CHUNK), jnp.int32),
            pltpu.VMEM((CHUNK, LATENT), jnp.float32),
            pltpu.VMEM_SHARED((NP_, LATENT), jnp.float32),
            pltpu.SemaphoreType.DMA,
        ],
        compiler_params=_SC_PARAMS,
    )
    def scat(idxr_hbm, val_hbm, zero_hbm, out_hbm, idx_v, buf, shared, sem):
        c = lax.axis_index("c")
        s = lax.axis_index("s")
        wid = s * 2 + c
        z0 = s * ROWS_PER_TILE
        pltpu.sync_copy(zero_hbm.at[pl.ds(z0, ROWS_PER_TILE)],
                        shared.at[pl.ds(z0, ROWS_PER_TILE)])
        plsc.subcore_barrier()
        row0 = wid * KCH
        pltpu.sync_copy(idxr_hbm.at[pl.ds(row0, KCH)], idx_v)

        def body(j, carry):
            base = (row0 + j) * CHUNK
            pltpu.sync_copy(val_hbm.at[pl.ds(base, CHUNK)], buf)
            pltpu.sync_copy(buf, shared.at[idx_v.at[j]], add=True)
            return carry

        lax.fori_loop(0, KCH, body, 0)
        plsc.subcore_barrier()
        pltpu.sync_copy(shared.at[pl.ds(z0, ROWS_PER_TILE)],
                        out_hbm.at[c, pl.ds(z0, ROWS_PER_TILE)])

    return scat


def _mlp_args(ps):
    out = []
    for (w, b) in ps:
        out.append(w)
        out.append(b.reshape(1, -1))
    return out


def kernel(position_sequence, n_particles_per_example, particle_types,
           senders, receivers, particle_type_embedding, params):
    del n_particles_per_example
    pf = position_sequence.reshape(N, SEQ * DIM)
    pf_p = jnp.pad(pf, ((0, NP_ - N), (0, 0)))
    tt_p = jnp.pad(particle_types.astype(jnp.int32),
                   (0, NP_ - N)).reshape(NP_, 1)
    emb_p = jnp.pad(particle_type_embedding, ((0, 16 - NTYPES), (0, 0)))
    snd_p = jnp.pad(senders.astype(jnp.int32),
                    (0, EP - E)).reshape(EP // CHUNK, CHUNK)
    rcv_p = jnp.pad(receivers.astype(jnp.int32), (0, EP - E),
                    constant_values=N).reshape(EP // CHUNK, CHUNK)
    postab = jnp.pad(pf[:, 15:18], ((0, NP_ - N), (0, 5)))
    zeros_n = jnp.zeros((NP_, LATENT), jnp.float32)

    nlat = _enc_node_call()(pf_p, tt_p, emb_p, *_mlp_args(params['enc_node']))
    ps, pr = _gather_call(8)(postab, snd_p, rcv_p)
    elat = _enc_edge_call()(ps, pr, *_mlp_args(params['enc_edge']))

    for s in range(5):
        gs, gr = _gather_call(64)(nlat, snd_p, rcv_p)
        (w1, b1), (w2, b2), (w3, b3) = params['proc_edge'][s]
        eupd, elat = _edge_step_call()(
            elat, gs, gr, w1[0:64], w1[64:128], w1[128:192],
            b1.reshape(1, -1), w2, b2.reshape(1, -1), w3, b3.reshape(1, -1))
        aggp = _scatter_call()(rcv_p, eupd, zeros_n)
        (v1, c1), (v2, c2), (v3, c3) = params['proc_node'][s]
        nlat = _node_step_call()(
            nlat, aggp, aggp, v1[0:64], v1[64:128], c1.reshape(1, -1),
            v2, c2.reshape(1, -1), v3, c3.reshape(1, -1))

    out = _dec_call()(nlat, pf_p, *_mlp_args(params['dec']))
    return out[:N]


# speedup vs baseline: 1.6804x; 1.0069x over previous
import functools

import jax
import jax.numpy as jnp
from jax import lax
from jax.experimental import pallas as pl
from jax.experimental.pallas import tpu as pltpu
from jax.experimental.pallas import tpu_sc as plsc

N = 10000
SEQ = 6
DIM = 3
LATENT = 64
NTYPES = 9
RADIUS = 0.5
E = 320000

NP_ = 10240
NW = 32
CHUNK = 128
KCH = 80
EP = NW * KCH * CHUNK
BE = 1024
BN = 256
ROWS_PER_TILE = NP_ // 16
NBUF = 2
NGRP = KCH // NBUF
D0 = 80


def _ln(y):
    m = jnp.mean(y, axis=-1, keepdims=True)
    d = y - m
    v = jnp.mean(d * d, axis=-1, keepdims=True)
    return d / jnp.sqrt(v + 1e-5)


def _dot(a, b):
    return jnp.dot(a, b, preferred_element_type=jnp.float32)


def _relu(x):
    return jnp.maximum(x, 0.0)


def _rows(bs, ncols):
    return pl.BlockSpec((bs, ncols), lambda i: (i, 0))


def _const(shape):
    return pl.BlockSpec(shape, lambda i: tuple(0 for _ in shape))


def _w64():
    return [_const((64, 64)), _const((1, 64))]


def _edge_w_specs():
    return [_const((64, 64))] * 3 + [_const((1, 64))] + 2 * _w64()


def _node_w_specs():
    return [_const((64, 64))] * 2 + [_const((1, 64))] + 2 * _w64()


def _enc_node_body(pf_ref, tt_ref, emb_ref, w1, b1, w2, b2, w3, b3, out_ref):
    pf = pf_ref[...]
    vel = pf[:, 3:18] - pf[:, 0:15]
    mr = pf[:, 15:18]
    ncb = jnp.clip(
        jnp.concatenate([mr, 1.0 - mr], axis=1) * (1.0 / RADIUS), -1.0, 1.0)
    oh = (lax.broadcasted_iota(jnp.int32, (BN, 16), 1) == tt_ref[...]).astype(
        jnp.float32)
    embv = _dot(oh, emb_ref[...])
    x = jnp.concatenate([vel, ncb, embv], axis=1)
    h = _relu(_dot(x, w1[...]) + b1[...])
    h = _relu(_dot(h, w2[...]) + b2[...])
    out_ref[...] = _ln(_dot(h, w3[...]) + b3[...])


def _mlp2relu_ln(x, b1, w2, b2, w3, b3):
    h = _relu(x + b1[...])
    h = _relu(_dot(h, w2[...]) + b2[...])
    return _ln(_dot(h, w3[...]) + b3[...])


def _edge0_body(gs_ref, gr_ref, ew1, eb1, ew2, eb2, ew3, eb3,
                w1e, w1s, w1r, b1, w2, b2, w3, b3, eupd_ref, elnew_ref):
    gs = gs_ref[...]
    gr = gr_ref[...]
    d = (gs[:, 64:72] - gr[:, 64:72]) * (1.0 / RADIUS)
    dist = jnp.sqrt(jnp.sum(d * d, axis=-1, keepdims=True))
    x = jnp.concatenate([d[:, 0:3], dist], axis=1)
    el = _mlp2relu_ln(_dot(x, ew1[...]), eb1, ew2, eb2, ew3, eb3)
    z = (_dot(el, w1e[...]) + _dot(gs[:, 0:64], w1s[...])
         + _dot(gr[:, 0:64], w1r[...]))
    e = _mlp2relu_ln(z, b1, w2, b2, w3, b3)
    eupd_ref[...] = e
    elnew_ref[...] = el + e


def _edge_step_body(el_ref, gs_ref, gr_ref, w1e, w1s, w1r, b1, w2, b2, w3, b3,
                    eupd_ref, elnew_ref):
    el = el_ref[...]
    z = (_dot(el, w1e[...]) + _dot(gs_ref[...], w1s[...])
         + _dot(gr_ref[...], w1r[...]))
    e = _mlp2relu_ln(z, b1, w2, b2, w3, b3)
    eupd_ref[...] = e
    elnew_ref[...] = el + e


def _node_step_body(nl_ref, a0_ref, a1_ref, w1n, w1a, b1, w2, b2, w3, b3,
                    out_ref):
    nl = nl_ref[...]
    agg = a0_ref[0] + a1_ref[0]
    z = _dot(nl, w1n[...]) + _dot(agg, w1a[...])
    out_ref[...] = nl + _mlp2relu_ln(z, b1, w2, b2, w3, b3)


def _node_dec_body(nl_ref, a0_ref, a1_ref, pf_ref,
                   w1n, w1a, b1, w2, b2, w3, b3,
                   dw1, db1, dw2, db2, dw3, db3, out_ref):
    nl = nl_ref[...]
    agg = a0_ref[0] + a1_ref[0]
    z = _dot(nl, w1n[...]) + _dot(agg, w1a[...])
    nl = nl + _mlp2relu_ln(z, b1, w2, b2, w3, b3)
    h = _relu(_dot(nl, dw1[...]) + db1[...])
    h = _relu(_dot(h, dw2[...]) + db2[...])
    acc = _dot(h, dw3[...]) + db3[...]
    pf = pf_ref[...]
    mr = pf[:, 15:18]
    prev = pf[:, 12:15]
    out_ref[...] = 2.0 * mr - prev + acc


def _mlp_specs(sizes):
    specs = []
    for i in range(len(sizes) - 1):
        specs.append(_const((sizes[i], sizes[i + 1])))
        specs.append(_const((1, sizes[i + 1])))
    return specs


@functools.lru_cache(maxsize=None)
def _enc_node_call():
    return pl.pallas_call(
        _enc_node_body,
        grid=(NP_ // BN,),
        in_specs=[_rows(BN, 18), _rows(BN, 1), _const((16, 16))]
        + _mlp_specs([37, 64, 64, 64]),
        out_specs=_rows(BN, LATENT),
        out_shape=jax.ShapeDtypeStruct((NP_, LATENT), jnp.float32),
    )


@functools.lru_cache(maxsize=None)
def _edge0_call():
    return pl.pallas_call(
        _edge0_body,
        grid=(EP // BE,),
        in_specs=[_rows(BE, D0), _rows(BE, D0)]
        + _mlp_specs([4, 64, 64, 64])
        + _edge_w_specs(),
        out_specs=(_rows(BE, LATENT), _rows(BE, LATENT)),
        out_shape=(jax.ShapeDtypeStruct((EP, LATENT), jnp.float32),
                   jax.ShapeDtypeStruct((EP, LATENT), jnp.float32)),
    )


@functools.lru_cache(maxsize=None)
def _edge_step_call():
    return pl.pallas_call(
        _edge_step_body,
        grid=(EP // BE,),
        in_specs=[_rows(BE, LATENT)] * 3 + _edge_w_specs(),
        out_specs=(_rows(BE, LATENT), _rows(BE, LATENT)),
        out_shape=(jax.ShapeDtypeStruct((EP, LATENT), jnp.float32),
                   jax.ShapeDtypeStruct((EP, LATENT), jnp.float32)),
    )


def _agg_specs():
    return [pl.BlockSpec((1, BN, LATENT), lambda i: (0, i, 0)),
            pl.BlockSpec((1, BN, LATENT), lambda i: (1, i, 0))]


@functools.lru_cache(maxsize=None)
def _node_step_call():
    return pl.pallas_call(
        _node_step_body,
        grid=(NP_ // BN,),
        in_specs=[_rows(BN, LATENT)] + _agg_specs() + _node_w_specs(),
        out_specs=_rows(BN, LATENT),
        out_shape=jax.ShapeDtypeStruct((NP_, LATENT), jnp.float32),
    )


@functools.lru_cache(maxsize=None)
def _node_dec_call():
    return pl.pallas_call(
        _node_dec_body,
        grid=(NP_ // BN,),
        in_specs=[_rows(BN, LATENT)] + _agg_specs() + [_rows(BN, 18)]
        + _node_w_specs() + _mlp_specs([64, 64, 64, 3]),
        out_specs=_rows(BN, 3),
        out_shape=jax.ShapeDtypeStruct((NP_, 3), jnp.float32),
    )


@functools.lru_cache(maxsize=None)
def _mesh():
    return plsc.VectorSubcoreMesh(
        core_axis_name="c", subcore_axis_name="s", num_cores=2,
        num_subcores=16)


_SC_PARAMS = pltpu.CompilerParams(use_tc_tiling_on_sc=False)


@functools.lru_cache(maxsize=None)
def _gather_call(d):
    @functools.partial(
        pl.kernel,
        out_type=(jax.ShapeDtypeStruct((EP, d), jnp.float32),
                  jax.ShapeDtypeStruct((EP, d), jnp.float32)),
        mesh=_mesh(),
        scratch_types=[
            pltpu.VMEM((KCH, CHUNK), jnp.int32),
            pltpu.VMEM((KCH, CHUNK), jnp.int32),
            pltpu.VMEM((2 * NBUF, CHUNK, d), jnp.float32),
            pltpu.VMEM((2 * NBUF, CHUNK, d), jnp.float32),
            pltpu.SemaphoreType.DMA,
            pltpu.SemaphoreType.DMA,
        ],
        compiler_params=_SC_PARAMS,
    )
    def gath(tab_hbm, idxs_hbm, idxr_hbm, outs_hbm, outr_hbm,
             idxs_v, idxr_v, bufs, bufr, semg, semw):
        wid = lax.axis_index("s") * 2 + lax.axis_index("c")
        row0 = wid * KCH
        pltpu.sync_copy(idxs_hbm.at[pl.ds(row0, KCH)], idxs_v)
        pltpu.sync_copy(idxr_hbm.at[pl.ds(row0, KCH)], idxr_v)

        def fire(g, st):
            for b in range(NBUF):
                j = g * NBUF + b
                pltpu.async_copy(tab_hbm.at[idxs_v.at[j]],
                                 bufs.at[st * NBUF + b], semg)
                pltpu.async_copy(tab_hbm.at[idxr_v.at[j]],
                                 bufr.at[st * NBUF + b], semg)

        def write(g, st):
            for b in range(NBUF):
                base = (row0 + g * NBUF + b) * CHUNK
                pltpu.async_copy(bufs.at[st * NBUF + b],
                                 outs_hbm.at[pl.ds(base, CHUNK)], semw)
                pltpu.async_copy(bufr.at[st * NBUF + b],
                                 outr_hbm.at[pl.ds(base, CHUNK)], semw)

        def drain_g():
            for _ in range(2 * NBUF):
                pltpu.make_async_copy(tab_hbm.at[pl.ds(0, CHUNK)],
                                      bufs.at[0], semg).wait()

        def drain_w():
            for _ in range(2 * NBUF):
                pltpu.make_async_copy(bufs.at[0],
                                      outs_hbm.at[pl.ds(0, CHUNK)],
                                      semw).wait()

        fire(0, 0)
        drain_g()
        fire(1, 1)
        write(0, 0)

        def body(p, carry):
            g = 2 * p + 1
            drain_g()
            drain_w()
            fire(g + 1, 0)
            write(g, 1)
            drain_g()
            drain_w()
            fire(g + 2, 1)
            write(g + 1, 0)
            return carry

        lax.fori_loop(0, (NGRP - 2) // 2, body, 0)
        g_last = NGRP - 1
        drain_g()
        drain_w()
        write(g_last, 1)
        drain_w()

    return gath


@functools.lru_cache(maxsize=None)
def _scatter_call():
    @functools.partial(
        pl.kernel,
        out_type=jax.ShapeDtypeStruct((2, NP_, LATENT), jnp.float32),
        mesh=_mesh(),
        scratch_types=[
            pltpu.VMEM((KCH, CHUNK), jnp.int32),
            pltpu.VMEM((2, CHUNK, LATENT), jnp.float32),
            pltpu.VMEM_SHARED((NP_, LATENT), jnp.float32),
            pltpu.SemaphoreType.DMA,
        ],
        compiler_params=_SC_PARAMS,
    )
    def scat(idxr_hbm, val_hbm, zero_hbm, out_hbm, idx_v, buf, shared, sem):
        c = lax.axis_index("c")
        s = lax.axis_index("s")
        wid = s * 2 + c
        z0 = s * ROWS_PER_TILE
        pltpu.sync_copy(zero_hbm.at[pl.ds(z0, ROWS_PER_TILE)],
                        shared.at[pl.ds(z0, ROWS_PER_TILE)])
        plsc.subcore_barrier()
        row0 = wid * KCH
        pltpu.sync_copy(idxr_hbm.at[pl.ds(row0, KCH)], idx_v)

        def load(j, b):
            pltpu.async_copy(val_hbm.at[pl.ds((row0 + j) * CHUNK, CHUNK)],
                             buf.at[b], sem)

        def drain():
            pltpu.make_async_copy(val_hbm.at[pl.ds(0, CHUNK)], buf.at[0],
                                  sem).wait()

        def scatter(j, b):
            pltpu.sync_copy(buf.at[b], shared.at[idx_v.at[j]], add=True)

        load(0, 0)

        def body(p, carry):
            j = 2 * p
            drain()
            load(j + 1, 1)
            scatter(j, 0)
            drain()
            load(j + 2, 0)
            scatter(j + 1, 1)
            return carry

        lax.fori_loop(0, KCH // 2 - 1, body, 0)
        j = KCH - 2
        drain()
        load(j + 1, 1)
        scatter(j, 0)
        drain()
        scatter(j + 1, 1)
        plsc.subcore_barrier()
        pltpu.sync_copy(shared.at[pl.ds(z0, ROWS_PER_TILE)],
                        out_hbm.at[c, pl.ds(z0, ROWS_PER_TILE)])

    return scat


def _mlp_args(ps):
    out = []
    for (w, b) in ps:
        out.append(w)
        out.append(b.reshape(1, -1))
    return out


def _edge_w(p):
    (w1, b1), (w2, b2), (w3, b3) = p
    return [w1[0:64], w1[64:128], w1[128:192], b1.reshape(1, -1),
            w2, b2.reshape(1, -1), w3, b3.reshape(1, -1)]


def _node_w(p):
    (v1, c1), (v2, c2), (v3, c3) = p
    return [v1[0:64], v1[64:128], c1.reshape(1, -1),
            v2, c2.reshape(1, -1), v3, c3.reshape(1, -1)]


def kernel(position_sequence, n_particles_per_example, particle_types,
           senders, receivers, particle_type_embedding, params):
    del n_particles_per_example
    pf = position_sequence.reshape(N, SEQ * DIM)
    pf_p = jnp.pad(pf, ((0, NP_ - N), (0, 0)))
    tt_p = jnp.pad(particle_types.astype(jnp.int32),
                   (0, NP_ - N)).reshape(NP_, 1)
    emb_p = jnp.pad(particle_type_embedding, ((0, 16 - NTYPES), (0, 0)))
    snd_p = jnp.pad(senders.astype(jnp.int32),
                    (0, EP - E)).reshape(EP // CHUNK, CHUNK)
    rcv_p = jnp.pad(receivers.astype(jnp.int32), (0, EP - E),
                    constant_values=N).reshape(EP // CHUNK, CHUNK)
    pos_p = jnp.pad(pf[:, 15:18], ((0, NP_ - N), (0, D0 - LATENT - DIM)))
    zeros_n = jnp.zeros((NP_, LATENT), jnp.float32)

    nlat = _enc_node_call()(pf_p, tt_p, emb_p, *_mlp_args(params['enc_node']))

    tab0 = jnp.concatenate([nlat, pos_p], axis=1)
    gs, gr = _gather_call(D0)(tab0, snd_p, rcv_p)
    eupd, elat = _edge0_call()(
        gs, gr, *_mlp_args(params['enc_edge']), *_edge_w(params['proc_edge'][0]))
    aggp = _scatter_call()(rcv_p, eupd, zeros_n)
    nlat = _node_step_call()(nlat, aggp, aggp, *_node_w(params['proc_node'][0]))

    for s in range(1, 5):
        gs, gr = _gather_call(LATENT)(nlat, snd_p, rcv_p)
        eupd, elat = _edge_step_call()(
            elat, gs, gr, *_edge_w(params['proc_edge'][s]))
        aggp = _scatter_call()(rcv_p, eupd, zeros_n)
        if s < 4:
            nlat = _node_step_call()(
                nlat, aggp, aggp, *_node_w(params['proc_node'][s]))
        else:
            out = _node_dec_call()(
                nlat, aggp, aggp, pf_p, *_node_w(params['proc_node'][s]),
                *_mlp_args(params['dec']))
    return out[:N]


# speedup vs baseline: 2.0660x; 1.2295x over previous
import functools

import jax
import jax.numpy as jnp
from jax import lax
from jax.experimental import pallas as pl
from jax.experimental.pallas import tpu as pltpu
from jax.experimental.pallas import tpu_sc as plsc

N = 10000
SEQ = 6
DIM = 3
LATENT = 64
NTYPES = 9
RADIUS = 0.5
E = 320000

NP_ = 10240
NW = 32
CHUNK = 128
KCH = 80
EP = NW * KCH * CHUNK
BE = 1024
BN = 256
ROWS_PER_TILE = NP_ // 16
W_EDGES = EP // NW
NBUF = 2
NGRP = KCH // NBUF
D0 = 80


def _ln(y):
    m = jnp.mean(y, axis=-1, keepdims=True)
    d = y - m
    v = jnp.mean(d * d, axis=-1, keepdims=True)
    return d / jnp.sqrt(v + 1e-5)


def _dot(a, b):
    return jnp.dot(a, b, preferred_element_type=jnp.float32)


def _relu(x):
    return jnp.maximum(x, 0.0)


def _rows(bs, ncols):
    return pl.BlockSpec((bs, ncols), lambda i: (i, 0))


def _const(shape):
    return pl.BlockSpec(shape, lambda i: tuple(0 for _ in shape))


def _w64():
    return [_const((64, 64)), _const((1, 64))]


def _edge_w_specs():
    return [_const((64, 64))] * 3 + [_const((1, 64))] + 2 * _w64()


def _node_w_specs():
    return [_const((64, 64))] * 2 + [_const((1, 64))] + 2 * _w64()


def _enc_node_body(pf_ref, tt_ref, emb_ref, w1, b1, w2, b2, w3, b3, out_ref):
    pf = pf_ref[...]
    vel = pf[:, 3:18] - pf[:, 0:15]
    mr = pf[:, 15:18]
    ncb = jnp.clip(
        jnp.concatenate([mr, 1.0 - mr], axis=1) * (1.0 / RADIUS), -1.0, 1.0)
    oh = (lax.broadcasted_iota(jnp.int32, (BN, 16), 1) == tt_ref[...]).astype(
        jnp.float32)
    embv = _dot(oh, emb_ref[...])
    x = jnp.concatenate([vel, ncb, embv], axis=1)
    h = _relu(_dot(x, w1[...]) + b1[...])
    h = _relu(_dot(h, w2[...]) + b2[...])
    out_ref[...] = _ln(_dot(h, w3[...]) + b3[...])


def _mlp2relu_ln(x, b1, w2, b2, w3, b3):
    h = _relu(x + b1[...])
    h = _relu(_dot(h, w2[...]) + b2[...])
    return _ln(_dot(h, w3[...]) + b3[...])


def _edge0_body(gs_ref, gr_ref, ew1, eb1, ew2, eb2, ew3, eb3,
                w1e, w1s, w1r, b1, w2, b2, w3, b3, eupd_ref, elnew_ref):
    gs = gs_ref[...]
    gr = gr_ref[...]
    es = []
    els = []
    for off in (0, 80):
        d = (gs[:, off + 64:off + 72] - gr[:, off + 64:off + 72]) * (
            1.0 / RADIUS)
        dist = jnp.sqrt(jnp.sum(d * d, axis=-1, keepdims=True))
        x = jnp.concatenate([d[:, 0:3], dist], axis=1)
        el = _mlp2relu_ln(_dot(x, ew1[...]), eb1, ew2, eb2, ew3, eb3)
        z = (_dot(el, w1e[...]) + _dot(gs[:, off:off + 64], w1s[...])
             + _dot(gr[:, off:off + 64], w1r[...]))
        es.append(_mlp2relu_ln(z, b1, w2, b2, w3, b3))
        els.append(el)
    e2 = jnp.concatenate(es, axis=1)
    eupd_ref[...] = e2
    elnew_ref[...] = jnp.concatenate(els, axis=1) + e2


def _edge_step_body(el_ref, gs_ref, gr_ref, w1e, w1s, w1r, b1, w2, b2, w3, b3,
                    eupd_ref, elnew_ref):
    el = el_ref[...]
    gs = gs_ref[...]
    gr = gr_ref[...]
    es = []
    for sl in (slice(0, 64), slice(64, 128)):
        z = (_dot(el[:, sl], w1e[...]) + _dot(gs[:, sl], w1s[...])
             + _dot(gr[:, sl], w1r[...]))
        es.append(_mlp2relu_ln(z, b1, w2, b2, w3, b3))
    e2 = jnp.concatenate(es, axis=1)
    eupd_ref[...] = e2
    elnew_ref[...] = el + e2


def _node_step_body(nl_ref, a0_ref, a1_ref, w1n, w1a, b1, w2, b2, w3, b3,
                    out_ref):
    nl = nl_ref[...]
    agg = a0_ref[0] + a1_ref[0]
    z = _dot(nl, w1n[...]) + _dot(agg, w1a[...])
    out_ref[...] = nl + _mlp2relu_ln(z, b1, w2, b2, w3, b3)


def _node_dec_body(nl_ref, a0_ref, a1_ref, pf_ref,
                   w1n, w1a, b1, w2, b2, w3, b3,
                   dw1, db1, dw2, db2, dw3, db3, out_ref):
    nl = nl_ref[...]
    agg = a0_ref[0] + a1_ref[0]
    z = _dot(nl, w1n[...]) + _dot(agg, w1a[...])
    nl = nl + _mlp2relu_ln(z, b1, w2, b2, w3, b3)
    h = _relu(_dot(nl, dw1[...]) + db1[...])
    h = _relu(_dot(h, dw2[...]) + db2[...])
    acc = _dot(h, dw3[...]) + db3[...]
    pf = pf_ref[...]
    mr = pf[:, 15:18]
    prev = pf[:, 12:15]
    out_ref[...] = 2.0 * mr - prev + acc


def _mlp_specs(sizes):
    specs = []
    for i in range(len(sizes) - 1):
        specs.append(_const((sizes[i], sizes[i + 1])))
        specs.append(_const((1, sizes[i + 1])))
    return specs


@functools.lru_cache(maxsize=None)
def _enc_node_call():
    return pl.pallas_call(
        _enc_node_body,
        grid=(NP_ // BN,),
        in_specs=[_rows(BN, 18), _rows(BN, 1), _const((16, 16))]
        + _mlp_specs([37, 64, 64, 64]),
        out_specs=_rows(BN, LATENT),
        out_shape=jax.ShapeDtypeStruct((NP_, LATENT), jnp.float32),
    )


@functools.lru_cache(maxsize=None)
def _edge0_call():
    return pl.pallas_call(
        _edge0_body,
        grid=(EP // BE,),
        in_specs=[_rows(BE // 2, 2 * D0), _rows(BE // 2, 2 * D0)]
        + _mlp_specs([4, 64, 64, 64])
        + _edge_w_specs(),
        out_specs=(_rows(BE // 2, 128), _rows(BE // 2, 128)),
        out_shape=(jax.ShapeDtypeStruct((EP // 2, 128), jnp.float32),
                   jax.ShapeDtypeStruct((EP // 2, 128), jnp.float32)),
    )


@functools.lru_cache(maxsize=None)
def _edge_step_call():
    return pl.pallas_call(
        _edge_step_body,
        grid=(EP // BE,),
        in_specs=[_rows(BE // 2, 128)] * 3 + _edge_w_specs(),
        out_specs=(_rows(BE // 2, 128), _rows(BE // 2, 128)),
        out_shape=(jax.ShapeDtypeStruct((EP // 2, 128), jnp.float32),
                   jax.ShapeDtypeStruct((EP // 2, 128), jnp.float32)),
    )


def _agg_specs():
    return [pl.BlockSpec((1, BN, LATENT), lambda i: (0, i, 0)),
            pl.BlockSpec((1, BN, LATENT), lambda i: (1, i, 0))]


@functools.lru_cache(maxsize=None)
def _node_step_call():
    return pl.pallas_call(
        _node_step_body,
        grid=(NP_ // BN,),
        in_specs=[_rows(BN, LATENT)] + _agg_specs() + _node_w_specs(),
        out_specs=_rows(BN, LATENT),
        out_shape=jax.ShapeDtypeStruct((NP_, LATENT), jnp.float32),
    )


@functools.lru_cache(maxsize=None)
def _node_dec_call():
    return pl.pallas_call(
        _node_dec_body,
        grid=(NP_ // BN,),
        in_specs=[_rows(BN, LATENT)] + _agg_specs() + [_rows(BN, 18)]
        + _node_w_specs() + _mlp_specs([64, 64, 64, 3]),
        out_specs=_rows(BN, 3),
        out_shape=jax.ShapeDtypeStruct((NP_, 3), jnp.float32),
    )


@functools.lru_cache(maxsize=None)
def _mesh():
    return plsc.VectorSubcoreMesh(
        core_axis_name="c", subcore_axis_name="s", num_cores=2,
        num_subcores=16)


_SC_PARAMS = pltpu.CompilerParams(use_tc_tiling_on_sc=False)


@functools.lru_cache(maxsize=None)
def _gather_call(d, dtype_name, t):
    dt = jnp.dtype(dtype_name)
    ngrp = W_EDGES // t

    @functools.partial(
        pl.kernel,
        out_type=(jax.ShapeDtypeStruct((EP, d), dt),
                  jax.ShapeDtypeStruct((EP, d), dt)),
        mesh=_mesh(),
        scratch_types=[
            pltpu.VMEM((W_EDGES // t, t), jnp.int32),
            pltpu.VMEM((W_EDGES // t, t), jnp.int32),
            pltpu.VMEM((2, t, d), dt),
            pltpu.VMEM((2, t, d), dt),
            pltpu.SemaphoreType.DMA,
            pltpu.SemaphoreType.DMA,
        ],
        compiler_params=_SC_PARAMS,
    )
    def gath(tab_hbm, idxs_hbm, idxr_hbm, outs_hbm, outr_hbm,
             idxs_v, idxr_v, bufs, bufr, semg, semw):
        wid = lax.axis_index("s") * 2 + lax.axis_index("c")
        row0 = wid * W_EDGES
        pltpu.sync_copy(idxs_hbm.at[wid], idxs_v)
        pltpu.sync_copy(idxr_hbm.at[wid], idxr_v)

        def fire(g, st):
            pltpu.async_copy(tab_hbm.at[idxs_v.at[g]], bufs.at[st], semg)
            pltpu.async_copy(tab_hbm.at[idxr_v.at[g]], bufr.at[st], semg)

        def write(g, st):
            base = row0 + g * t
            pltpu.async_copy(bufs.at[st],
                             outs_hbm.at[pl.ds(base, t)], semw)
            pltpu.async_copy(bufr.at[st],
                             outr_hbm.at[pl.ds(base, t)], semw)

        def drain_g():
            for _ in range(2):
                pltpu.make_async_copy(tab_hbm.at[pl.ds(0, t)],
                                      bufs.at[0], semg).wait()

        def drain_w():
            for _ in range(2):
                pltpu.make_async_copy(bufs.at[0],
                                      outs_hbm.at[pl.ds(0, t)],
                                      semw).wait()

        fire(0, 0)
        drain_g()
        fire(1, 1)
        write(0, 0)

        def body(p, carry):
            g = 2 * p + 1
            drain_g()
            drain_w()
            fire(g + 1, 0)
            write(g, 1)
            drain_g()
            drain_w()
            fire(g + 2, 1)
            write(g + 1, 0)
            return carry

        lax.fori_loop(0, (ngrp - 2) // 2, body, 0)
        g_last = ngrp - 1
        drain_g()
        drain_w()
        write(g_last, 1)
        drain_w()

    return gath


@functools.lru_cache(maxsize=None)
def _scatter_call():
    @functools.partial(
        pl.kernel,
        out_type=jax.ShapeDtypeStruct((2, NP_, LATENT), jnp.float32),
        mesh=_mesh(),
        scratch_types=[
            pltpu.VMEM((W_EDGES // 512, 512), jnp.int32),
            pltpu.VMEM((2, 512, LATENT), jnp.float32),
            pltpu.VMEM_SHARED((NP_, LATENT), jnp.float32),
            pltpu.SemaphoreType.DMA,
        ],
        compiler_params=_SC_PARAMS,
    )
    def scat(idxr_hbm, val_hbm, zero_hbm, out_hbm, idx_v, buf, shared, sem):
        c = lax.axis_index("c")
        s = lax.axis_index("s")
        wid = s * 2 + c
        z0 = s * ROWS_PER_TILE
        pltpu.sync_copy(zero_hbm.at[pl.ds(z0, ROWS_PER_TILE)],
                        shared.at[pl.ds(z0, ROWS_PER_TILE)])
        plsc.subcore_barrier()
        row0 = wid * W_EDGES
        pltpu.sync_copy(idxr_hbm.at[wid], idx_v)
        ngs = W_EDGES // 512

        def load(g, b):
            pltpu.async_copy(val_hbm.at[pl.ds(row0 + g * 512, 512)],
                             buf.at[b], sem)

        def drain():
            pltpu.make_async_copy(val_hbm.at[pl.ds(0, 512)], buf.at[0],
                                  sem).wait()

        def scatter(g, b):
            pltpu.sync_copy(buf.at[b], shared.at[idx_v.at[g]], add=True)

        load(0, 0)

        def body(p, carry):
            g = 2 * p
            drain()
            load(g + 1, 1)
            scatter(g, 0)
            drain()
            load(g + 2, 0)
            scatter(g + 1, 1)
            return carry

        lax.fori_loop(0, ngs // 2 - 1, body, 0)
        g = ngs - 2
        drain()
        load(g + 1, 1)
        scatter(g, 0)
        drain()
        scatter(g + 1, 1)
        plsc.subcore_barrier()
        pltpu.sync_copy(shared.at[pl.ds(z0, ROWS_PER_TILE)],
                        out_hbm.at[c, pl.ds(z0, ROWS_PER_TILE)])

    return scat


def _mlp_args(ps):
    out = []
    for (w, b) in ps:
        out.append(w)
        out.append(b.reshape(1, -1))
    return out


def _edge_w(p):
    (w1, b1), (w2, b2), (w3, b3) = p
    return [w1[0:64], w1[64:128], w1[128:192], b1.reshape(1, -1),
            w2, b2.reshape(1, -1), w3, b3.reshape(1, -1)]


def _edge_w_bf(p):
    (w1, b1), (w2, b2), (w3, b3) = p
    bf = jnp.bfloat16
    return [w1[0:64].astype(bf), w1[64:128].astype(bf),
            w1[128:192].astype(bf), b1.reshape(1, -1),
            w2, b2.reshape(1, -1), w3, b3.reshape(1, -1)]


def _node_w(p):
    (v1, c1), (v2, c2), (v3, c3) = p
    return [v1[0:64], v1[64:128], c1.reshape(1, -1),
            v2, c2.reshape(1, -1), v3, c3.reshape(1, -1)]


def kernel(position_sequence, n_particles_per_example, particle_types,
           senders, receivers, particle_type_embedding, params):
    del n_particles_per_example
    pf = position_sequence.reshape(N, SEQ * DIM)
    pf_p = jnp.pad(pf, ((0, NP_ - N), (0, 0)))
    tt_p = jnp.pad(particle_types.astype(jnp.int32),
                   (0, NP_ - N)).reshape(NP_, 1)
    emb_p = jnp.pad(particle_type_embedding, ((0, 16 - NTYPES), (0, 0)))
    snd_pad = jnp.pad(senders.astype(jnp.int32), (0, EP - E))
    rcv_pad = jnp.pad(receivers.astype(jnp.int32), (0, EP - E),
                      constant_values=N)
    snd_256 = snd_pad.reshape(NW, W_EDGES // 256, 256)
    rcv_256 = rcv_pad.reshape(NW, W_EDGES // 256, 256)
    snd_512 = snd_pad.reshape(NW, W_EDGES // 512, 512)
    rcv_512 = rcv_pad.reshape(NW, W_EDGES // 512, 512)
    pos_p = jnp.pad(pf[:, 15:18], ((0, NP_ - N), (0, D0 - LATENT - DIM)))
    zeros_n = jnp.zeros((NP_, LATENT), jnp.float32)

    nlat = _enc_node_call()(pf_p, tt_p, emb_p, *_mlp_args(params['enc_node']))

    tab0 = jnp.concatenate([nlat, pos_p], axis=1)
    gs, gr = _gather_call(D0, 'float32', 256)(tab0, snd_256, rcv_256)
    eupd, elat = _edge0_call()(
        gs.reshape(EP // 2, 2 * D0), gr.reshape(EP // 2, 2 * D0),
        *_mlp_args(params['enc_edge']), *_edge_w(params['proc_edge'][0]))
    aggp = _scatter_call()(rcv_512, eupd.reshape(EP, LATENT), zeros_n)
    nlat = _node_step_call()(
        nlat, aggp, aggp, *_node_w(params['proc_node'][0]))

    for s in range(1, 5):
        gs, gr = _gather_call(LATENT, 'float32', 256)(nlat, snd_256, rcv_256)
        eupd, elat = _edge_step_call()(
            elat, gs.reshape(EP // 2, 128), gr.reshape(EP // 2, 128),
            *_edge_w(params['proc_edge'][s]))
        aggp = _scatter_call()(rcv_512, eupd.reshape(EP, LATENT), zeros_n)
        if s < 4:
            nlat = _node_step_call()(
                nlat, aggp, aggp, *_node_w(params['proc_node'][s]))
        else:
            out = _node_dec_call()(
                nlat, aggp, aggp, pf_p, *_node_w(params['proc_node'][s]),
                *_mlp_args(params['dec']))
    return out[:N]
